# Initial kernel scaffold; baseline (speedup 1.0000x reference)
import jax
import jax.numpy as jnp
from jax.experimental import pallas as pl


def kernel(x):
    raise NotImplementedError("write your pallas kernel here")


# speedup vs baseline: 31.0513x; 31.0513x over previous
import dataclasses
import functools

import jax
import jax.numpy as jnp
from jax import lax
from jax.experimental import pallas as pl
from jax.experimental.pallas import tpu as pltpu
from jax.experimental.pallas import tpu_sc as plsc

_N = 2048
_P = _N * (_N - 1) // 2
_NW = 32
_PW = _P // _NW
_NBLK = 16
_BLK_MAIN = 4096
_BLK_TAIL = _PW - (_NBLK - 1) * _BLK_MAIN
_XPAD = _N + 16
_STAGE = 2 * (_BLK_MAIN + 16)


def _pairs_kernel_body(x_hbm, out_hbm, x_v, stage):
    wid = lax.axis_index("s") * 2 + lax.axis_index("c")
    pltpu.sync_copy(x_hbm, x_v.at[pl.ds(0, _N)])

    p0 = wid * _PW

    def _off(i):
        return (i * (2 * _N - 1 - i)) // 2

    def _bs_body(_, lohi):
        lo, hi = lohi
        mid = (lo + hi + 1) // 2
        take = _off(mid) <= p0
        return (jnp.where(take, mid, lo), jnp.where(take, hi, mid - 1))

    i0, _ = lax.fori_loop(0, 11, _bs_body, (jnp.int32(0), jnp.int32(_N - 1)))
    j0 = i0 + 1 + (p0 - _off(i0))

    lanes = lax.iota(jnp.int32, 16)

    def emit_block(i, j, blk, base):
        def cond(c):
            return c[2] < blk

        def body(c):
            ci, cj, q = c
            l_seg = _N - cj
            step = jnp.minimum(jnp.minimum(16, l_seg), blk - q)
            b = x_v[pl.ds(cj, 16)]
            a = plsc.load_gather(x_v, [jnp.full((16,), ci, jnp.int32)])
            ev = 2 * (q + lanes)
            plsc.store_scatter(stage, [ev], a)
            plsc.store_scatter(stage, [ev + 1], b)
            done = step == l_seg
            ni = jnp.where(done, ci + 1, ci)
            nj = jnp.where(done, ci + 2, cj + step)
            return (ni, nj, q + step)

        i, j, _ = lax.while_loop(cond, body, (i, j, jnp.int32(0)))
        pltpu.sync_copy(stage.at[pl.ds(0, 2 * blk)],
                        out_hbm.at[pl.ds(base, 2 * blk)])
        return i, j

    i, j = i0, j0
    obase = wid * (2 * _PW)
    for k in range(_NBLK):
        blk = _BLK_MAIN if k < _NBLK - 1 else _BLK_TAIL
        i, j = emit_block(i, j, blk, obase + k * 2 * _BLK_MAIN)


@jax.jit
def kernel(x):
    mesh = plsc.VectorSubcoreMesh(core_axis_name="c", subcore_axis_name="s")
    cp = pltpu.CompilerParams()
    if "needs_layout_passes" in pltpu.CompilerParams.__dataclass_fields__:
        cp = dataclasses.replace(cp, needs_layout_passes=False)
    run = pl.kernel(
        _pairs_kernel_body,
        out_type=jax.ShapeDtypeStruct((2 * _P,), jnp.float32),
        mesh=mesh,
        compiler_params=cp,
        scratch_types=[
            pltpu.VMEM((_XPAD,), jnp.float32),
            pltpu.VMEM((_STAGE,), jnp.float32),
        ],
    )
    return run(x).reshape(_P, 2)


# speedup vs baseline: 31.4742x; 1.0136x over previous
import dataclasses
import functools

import jax
import jax.numpy as jnp
from jax import lax
from jax.experimental import pallas as pl
from jax.experimental.pallas import tpu as pltpu
from jax.experimental.pallas import tpu_sc as plsc

_N = 2048
_P = _N * (_N - 1) // 2
_NW = 32
_PW = _P // _NW
_NBLK = 4
_BLK = _PW // _NBLK
_XPAD = _N + 16
_STAGE = 2 * (_BLK + 16)


def _pairs_kernel_body(x_hbm, out_hbm, x_v, stage0, stage1, sem0, sem1):
    wid = lax.axis_index("s") * 2 + lax.axis_index("c")
    pltpu.sync_copy(x_hbm, x_v.at[pl.ds(0, _N)])

    p0 = wid * _PW

    def _off(i):
        return (i * (2 * _N - 1 - i)) // 2

    def _bs_body(_, lohi):
        lo, hi = lohi
        mid = (lo + hi + 1) // 2
        take = _off(mid) <= p0
        return (jnp.where(take, mid, lo), jnp.where(take, hi, mid - 1))

    i0, _ = lax.fori_loop(0, 11, _bs_body, (jnp.int32(0), jnp.int32(_N - 1)))
    j0 = i0 + 1 + (p0 - _off(i0))

    lanes2 = 2 * lax.iota(jnp.int32, 16)
    stages = (stage0, stage1)
    sems = (sem0, sem1)

    def fill_block(i, j, stage):
        def cond(c):
            return c[2] < _BLK

        def body(c):
            ci, cj, q = c
            l_seg = _N - cj
            m = jnp.minimum(l_seg, _BLK - q)
            a = plsc.load_gather(x_v, [jnp.full((16,), ci, jnp.int32)])
            nch = (m + 15) // 16

            def chunk(t, cc):
                jj, evv = cc
                b = x_v[pl.ds(jj, 16)]
                plsc.store_scatter(stage, [evv], a)
                plsc.store_scatter(stage, [evv + 1], b)
                return (jj + 16, evv + 32)

            lax.fori_loop(0, nch, chunk, (cj, 2 * q + lanes2))
            done = m == l_seg
            ni = jnp.where(done, ci + 1, ci)
            nj = jnp.where(done, ci + 2, cj + m)
            return (ni, nj, q + m)

        i, j, _ = lax.while_loop(cond, body, (i, j, jnp.int32(0)))
        return i, j

    i, j = i0, j0
    obase = wid * (2 * _PW)
    copies = [None, None]
    for k in range(_NBLK):
        b = k % 2
        if copies[b] is not None:
            copies[b].wait()
        i, j = fill_block(i, j, stages[b])
        copies[b] = pltpu.make_async_copy(
            stages[b].at[pl.ds(0, 2 * _BLK)],
            out_hbm.at[pl.ds(obase + k * 2 * _BLK, 2 * _BLK)],
            sems[b],
        )
        copies[b].start()
    copies[0].wait()
    copies[1].wait()


@jax.jit
def kernel(x):
    mesh = plsc.VectorSubcoreMesh(core_axis_name="c", subcore_axis_name="s")
    cp = pltpu.CompilerParams()
    if "needs_layout_passes" in pltpu.CompilerParams.__dataclass_fields__:
        cp = dataclasses.replace(cp, needs_layout_passes=False)
    run = pl.kernel(
        _pairs_kernel_body,
        out_type=jax.ShapeDtypeStruct((2 * _P,), jnp.float32),
        mesh=mesh,
        compiler_params=cp,
        scratch_types=[
            pltpu.VMEM((_XPAD,), jnp.float32),
            pltpu.VMEM((_STAGE,), jnp.float32),
            pltpu.VMEM((_STAGE,), jnp.float32),
            pltpu.SemaphoreType.DMA,
            pltpu.SemaphoreType.DMA,
        ],
    )
    return run(x).reshape(_P, 2)


# speedup vs baseline: 720.3667x; 22.8876x over previous
import dataclasses
import functools

import jax
import jax.numpy as jnp
from jax import lax
from jax.experimental import pallas as pl
from jax.experimental.pallas import tpu as pltpu
from jax.experimental.pallas import tpu_sc as plsc

_N = 2048
_P = _N * (_N - 1) // 2
_NW = 32
_PW = _P // _NW
_NBLK = 4
_BLK = _PW // _NBLK
_XPAD = _N + 16
_STAGE = _BLK + 16


def _pairs_kernel_body(x_hbm, c0_hbm, c1_hbm, x_v,
                       a0, b0, a1, b1, sem0, sem1):
    wid = lax.axis_index("s") * 2 + lax.axis_index("c")
    pltpu.sync_copy(x_hbm, x_v.at[pl.ds(0, _N)])

    p0 = wid * _PW

    def _off(i):
        return (i * (2 * _N - 1 - i)) // 2

    def _bs_body(_, lohi):
        lo, hi = lohi
        mid = (lo + hi + 1) // 2
        take = _off(mid) <= p0
        return (jnp.where(take, mid, lo), jnp.where(take, hi, mid - 1))

    i0, _ = lax.fori_loop(0, 11, _bs_body, (jnp.int32(0), jnp.int32(_N - 1)))
    j0 = i0 + 1 + (p0 - _off(i0))

    bufs = ((a0, b0), (a1, b1))
    sems = (sem0, sem1)

    def fill_block(i, j, sa, sb):
        def cond(c):
            return c[2] < _BLK

        def body(c):
            ci, cj, q = c
            l_seg = _N - cj
            m = jnp.minimum(l_seg, _BLK - q)
            a = plsc.load_gather(x_v, [jnp.full((16,), ci, jnp.int32)])

            def chunk(t, cc):
                jj, qq = cc
                sa[pl.ds(qq, 16)] = a
                sb[pl.ds(qq, 16)] = x_v[pl.ds(jj, 16)]
                return (jj + 16, qq + 16)

            lax.fori_loop(0, (m + 15) // 16, chunk, (cj, q))
            done = m == l_seg
            ni = jnp.where(done, ci + 1, ci)
            nj = jnp.where(done, ci + 2, cj + m)
            return (ni, nj, q + m)

        i, j, _ = lax.while_loop(cond, body, (i, j, jnp.int32(0)))
        return i, j

    i, j = i0, j0
    base = wid * _PW
    copies = [None, None]
    for k in range(_NBLK):
        b = k % 2
        if copies[b] is not None:
            copies[b][0].wait()
            copies[b][1].wait()
        sa, sb = bufs[b]
        i, j = fill_block(i, j, sa, sb)
        dst = pl.ds(base + k * _BLK, _BLK)
        ca = pltpu.make_async_copy(sa.at[pl.ds(0, _BLK)], c0_hbm.at[dst],
                                   sems[b])
        cb = pltpu.make_async_copy(sb.at[pl.ds(0, _BLK)], c1_hbm.at[dst],
                                   sems[b])
        ca.start()
        cb.start()
        copies[b] = (ca, cb)
    copies[0][0].wait()
    copies[0][1].wait()
    copies[1][0].wait()
    copies[1][1].wait()


@jax.jit
def kernel(x):
    mesh = plsc.VectorSubcoreMesh(core_axis_name="c", subcore_axis_name="s")
    cp = pltpu.CompilerParams()
    if "needs_layout_passes" in pltpu.CompilerParams.__dataclass_fields__:
        cp = dataclasses.replace(cp, needs_layout_passes=False)
    run = pl.kernel(
        _pairs_kernel_body,
        out_type=(jax.ShapeDtypeStruct((_P,), jnp.float32),
                  jax.ShapeDtypeStruct((_P,), jnp.float32)),
        mesh=mesh,
        compiler_params=cp,
        scratch_types=[
            pltpu.VMEM((_XPAD,), jnp.float32),
            pltpu.VMEM((_STAGE,), jnp.float32),
            pltpu.VMEM((_STAGE,), jnp.float32),
            pltpu.VMEM((_STAGE,), jnp.float32),
            pltpu.VMEM((_STAGE,), jnp.float32),
            pltpu.SemaphoreType.DMA,
            pltpu.SemaphoreType.DMA,
        ],
    )
    c0, c1 = run(x)
    return jnp.stack([c0, c1], axis=-1)


# speedup vs baseline: 847.1054x; 1.1759x over previous
import dataclasses
import functools

import jax
import jax.numpy as jnp
from jax import lax
from jax.experimental import pallas as pl
from jax.experimental.pallas import tpu as pltpu
from jax.experimental.pallas import tpu_sc as plsc

_N = 2048
_P = _N * (_N - 1) // 2
_NW = 32
_PW = _P // _NW
_NBLK = 4
_BLK = _PW // _NBLK
_XPAD = _N + 64
_STAGE = _BLK + 64


def _pairs_kernel_body(x_hbm, c0_hbm, c1_hbm, x_v,
                       a0, b0, a1, b1, sem0, sem1):
    wid = lax.axis_index("s") * 2 + lax.axis_index("c")
    pltpu.sync_copy(x_hbm, x_v.at[pl.ds(0, _N)])

    p0 = wid * _PW

    def _off(i):
        return (i * (2 * _N - 1 - i)) // 2

    def _bs_body(_, lohi):
        lo, hi = lohi
        mid = (lo + hi + 1) // 2
        take = _off(mid) <= p0
        return (jnp.where(take, mid, lo), jnp.where(take, hi, mid - 1))

    i0, _ = lax.fori_loop(0, 11, _bs_body, (jnp.int32(0), jnp.int32(_N - 1)))
    j0 = i0 + 1 + (p0 - _off(i0))

    bufs = ((a0, b0), (a1, b1))
    sems = (sem0, sem1)

    def fill_block(i, j, sa, sb):
        def cond(c):
            return c[2] < _BLK

        def body(c):
            ci, cj, q = c
            l_seg = _N - cj
            m = jnp.minimum(l_seg, _BLK - q)
            a = plsc.load_gather(x_v, [jnp.full((16,), ci, jnp.int32)])

            def chunk(t, cc):
                jj, qq = cc
                b1 = x_v[pl.ds(jj, 16)]
                b2 = x_v[pl.ds(jj + 16, 16)]
                b3 = x_v[pl.ds(jj + 32, 16)]
                b4 = x_v[pl.ds(jj + 48, 16)]
                sa[pl.ds(qq, 16)] = a
                sa[pl.ds(qq + 16, 16)] = a
                sa[pl.ds(qq + 32, 16)] = a
                sa[pl.ds(qq + 48, 16)] = a
                sb[pl.ds(qq, 16)] = b1
                sb[pl.ds(qq + 16, 16)] = b2
                sb[pl.ds(qq + 32, 16)] = b3
                sb[pl.ds(qq + 48, 16)] = b4
                return (jj + 64, qq + 64)

            lax.fori_loop(0, (m + 63) // 64, chunk, (cj, q))
            done = m == l_seg
            ni = jnp.where(done, ci + 1, ci)
            nj = jnp.where(done, ci + 2, cj + m)
            return (ni, nj, q + m)

        i, j, _ = lax.while_loop(cond, body, (i, j, jnp.int32(0)))
        return i, j

    i, j = i0, j0
    base = wid * _PW
    copies = [None, None]
    for k in range(_NBLK):
        b = k % 2
        if copies[b] is not None:
            copies[b][0].wait()
            copies[b][1].wait()
        sa, sb = bufs[b]
        i, j = fill_block(i, j, sa, sb)
        dst = pl.ds(base + k * _BLK, _BLK)
        ca = pltpu.make_async_copy(sa.at[pl.ds(0, _BLK)], c0_hbm.at[dst],
                                   sems[b])
        cb = pltpu.make_async_copy(sb.at[pl.ds(0, _BLK)], c1_hbm.at[dst],
                                   sems[b])
        ca.start()
        cb.start()
        copies[b] = (ca, cb)
    copies[0][0].wait()
    copies[0][1].wait()
    copies[1][0].wait()
    copies[1][1].wait()


@jax.jit
def kernel(x):
    mesh = plsc.VectorSubcoreMesh(core_axis_name="c", subcore_axis_name="s")
    cp = pltpu.CompilerParams()
    if "needs_layout_passes" in pltpu.CompilerParams.__dataclass_fields__:
        cp = dataclasses.replace(cp, needs_layout_passes=False)
    run = pl.kernel(
        _pairs_kernel_body,
        out_type=(jax.ShapeDtypeStruct((_P,), jnp.float32),
                  jax.ShapeDtypeStruct((_P,), jnp.float32)),
        mesh=mesh,
        compiler_params=cp,
        scratch_types=[
            pltpu.VMEM((_XPAD,), jnp.float32),
            pltpu.VMEM((_STAGE,), jnp.float32),
            pltpu.VMEM((_STAGE,), jnp.float32),
            pltpu.VMEM((_STAGE,), jnp.float32),
            pltpu.VMEM((_STAGE,), jnp.float32),
            pltpu.SemaphoreType.DMA,
            pltpu.SemaphoreType.DMA,
        ],
    )
    c0, c1 = run(x)
    return jnp.stack([c0, c1], axis=-1)
